# Initial kernel scaffold; baseline (speedup 1.0000x reference)
import jax
import jax.numpy as jnp
from jax.experimental import pallas as pl


def kernel(x, pos_table, sec_table, W, b):
    raise NotImplementedError("write your pallas kernel here")


# speedup vs baseline: 4.9900x; 4.9900x over previous
import functools

import jax
import jax.numpy as jnp
from jax import lax
from jax.experimental import pallas as pl
from jax.experimental.pallas import tpu as pltpu
from jax.experimental.pallas import tpu_sc as plsc


def _make_sc_gather(V, E, B):
    info = plsc.get_sparse_core_info()
    NC, NS = info.num_cores, info.num_subcores
    NW = NC * NS
    G = 128
    C = 1024
    assert B % (NW * C) == 0
    b_per_w = B // NW
    n_chunks = b_per_w // C
    mesh = plsc.VectorSubcoreMesh(core_axis_name="c", subcore_axis_name="s")

    @functools.partial(
        pl.kernel,
        mesh=mesh,
        compiler_params=pltpu.CompilerParams(use_tc_tiling_on_sc=False),
        out_type=jax.ShapeDtypeStruct((B, E), jnp.float32),
        scratch_types=[
            pltpu.VMEM((C,), jnp.int32),
            pltpu.VMEM((C, E), jnp.float32),
            pltpu.VMEM_SHARED((V, E), jnp.float32),
            pltpu.SemaphoreType.DMA,
        ],
    )
    def sc_gather(table_hbm, idx_hbm, out_hbm, idx_v, rows_v, table_sh, sem):
        sid = lax.axis_index("s")
        wid = sid * NC + lax.axis_index("c")
        base = wid * b_per_w

        @pl.when(sid == 0)
        def _stage():
            pltpu.sync_copy(table_hbm, table_sh)

        plsc.subcore_barrier()

        def chunk(c, carry):
            tok0 = base + c * C
            pltpu.sync_copy(idx_hbm.at[pl.ds(tok0, C)], idx_v)
            copies = []
            for j in range(C // G):
                copies.append(pltpu.async_copy(
                    table_sh.at[idx_v.at[pl.ds(j * G, G)]],
                    rows_v.at[pl.ds(j * G, G)],
                    sem,
                ))
            for cp in copies:
                cp.wait()
            pltpu.sync_copy(rows_v, out_hbm.at[pl.ds(tok0, C)])
            return carry

        lax.fori_loop(0, n_chunks, chunk, 0)

    return sc_gather


def _tc_body(x_ref, sec_ref, wt_ref, posb_ref, out_ref):
    acc0 = jnp.dot(wt_ref[...], x_ref[0], preferred_element_type=jnp.float32)
    acc1 = jnp.dot(wt_ref[...], x_ref[1], preferred_element_type=jnp.float32)
    sec_t = jnp.transpose(sec_ref[...])
    out_ref[0] = acc0 + sec_t[0:64, :] + posb_ref[0]
    out_ref[1] = acc1 + sec_t[64:128, :] + posb_ref[1]


def _tc_dense(xt, sec128, wt, posb3, BN):
    P, D, N = xt.shape
    E = wt.shape[0]
    nb = N // BN
    return pl.pallas_call(
        _tc_body,
        grid=(P // 2, nb),
        in_specs=[
            pl.BlockSpec((2, D, BN), lambda q, j: (q, 0, j)),
            pl.BlockSpec((BN, 128), lambda q, j: (q * nb + j, 0)),
            pl.BlockSpec((E, D), lambda q, j: (0, 0)),
            pl.BlockSpec((2, E, 1), lambda q, j: (q, 0, 0)),
        ],
        out_specs=pl.BlockSpec((2, E, BN), lambda q, j: (q, 0, j)),
        out_shape=jax.ShapeDtypeStruct((P, E, N), jnp.float32),
    )(xt, sec128, wt, posb3)


def kernel(x, pos_table, sec_table, W, b):
    N, P, D = x.shape
    V, E = sec_table.shape
    B = N * P

    xt = jnp.transpose(x, (1, 2, 0))
    idx_t = xt[:, 0, :].astype(jnp.int32)
    idx_sc = jnp.transpose(idx_t.reshape(P // 2, 2, N), (0, 2, 1)).reshape(B)

    positions = jnp.take(pos_table, jnp.arange(D - 1, -1, -1), axis=0)
    posb3 = (positions + b[None, :]).reshape(D, E, 1)
    wt = jnp.concatenate([jnp.zeros((1, E), W.dtype), W], axis=0).T

    sec = _make_sc_gather(V, E, B)(sec_table, idx_sc)
    sec128 = sec.reshape(B // 2, 2 * E)
    out_t = _tc_dense(xt, sec128, wt, posb3, 2048)
    return jnp.transpose(out_t, (2, 0, 1))
